# baseline (speedup 1.0000x reference)
import jax
import jax.numpy as jnp
from jax.experimental import pallas as pl
from jax.experimental.pallas import tpu as pltpu

_IN = 12
_H = 10
_PK = 32
_KW = _PK * _IN
_HW = _PK * _H


def _body(x_ref, w1e_ref, aux_ref, w2e_ref, o_ref):
    b1e = aux_ref[:, :_HW]
    b2 = aux_ref[0:1, _HW:_HW + 1]
    h = jnp.dot(x_ref[...], w1e_ref[...], preferred_element_type=jnp.float32)
    h = jnp.tanh(h + b1e)
    o = jnp.dot(h, w2e_ref[...], preferred_element_type=jnp.float32)
    o_ref[...] = o + b2


def _expand_params(w1, b1, w2, b2):
    eye = jnp.eye(_PK, dtype=jnp.float32)
    w1e = jnp.kron(eye, w1.T.astype(jnp.float32))
    w2e = jnp.kron(eye, w2.astype(jnp.float32).reshape(_H, 1))
    aux = jnp.zeros((1, _KW), jnp.float32)
    aux = aux.at[0, :_HW].set(jnp.tile(b1.astype(jnp.float32), _PK))
    aux = aux.at[0, _HW].set(b2.astype(jnp.float32)[0])
    return w1e, aux, w2e


def kernel(x, w1, b1, w2, b2):
    B = x.shape[0]
    x = x.astype(jnp.float32)
    if B % _PK:
        return _kernel_rowwise(x, w1, b1, w2, b2, B)
    rows = B // _PK

    tb = 1024
    while rows % tb:
        tb //= 2
    if tb < 8:
        return _kernel_rowwise(x, w1, b1, w2, b2, B)

    w1e, aux, w2e = _expand_params(w1, b1, w2, b2)
    xp = x.reshape(rows, _KW)

    out = pl.pallas_call(
        _body,
        out_shape=jax.ShapeDtypeStruct((rows, _PK), jnp.float32),
        grid=(rows // tb,),
        in_specs=[
            pl.BlockSpec((tb, _KW), lambda i: (i, 0)),
            pl.BlockSpec((_KW, _HW), lambda i: (0, 0)),
            pl.BlockSpec((1, _KW), lambda i: (0, 0)),
            pl.BlockSpec((_HW, _PK), lambda i: (0, 0)),
        ],
        out_specs=pl.BlockSpec((tb, _PK), lambda i: (i, 0)),
        compiler_params=pltpu.CompilerParams(
            dimension_semantics=("parallel",),
        ),
    )(xp, w1e, aux, w2e)

    return out.reshape(B, 1)


_CB1 = _IN
_CW2 = _IN + 1
_CB2 = _IN + 2
_PC = _IN + 3


def _rowwise_body(x_ref, p_ref, o_ref):
    w1 = p_ref[:, :_IN]
    b1 = p_ref[:, _CB1:_CB1 + 1]
    w2 = p_ref[:, _CW2:_CW2 + 1]
    b2 = p_ref[0:1, _CB2:_CB2 + 1]
    ht = jax.lax.dot_general(
        w1, x_ref[...], (((1,), (1,)), ((), ())),
        preferred_element_type=jnp.float32,
    )
    ht = jnp.tanh(ht + b1)
    o_ref[...] = jnp.sum(ht * w2, axis=0, keepdims=True) + b2


def _kernel_rowwise(x, w1, b1, w2, b2, B):
    p = jnp.zeros((_H, _PC), jnp.float32)
    p = p.at[:, :_IN].set(w1.astype(jnp.float32))
    p = p.at[:, _CB1].set(b1.astype(jnp.float32))
    p = p.at[:, _CW2].set(w2[0].astype(jnp.float32))
    p = p.at[0, _CB2].set(b2[0].astype(jnp.float32))
    tb = B
    for cand in (512, 256, 128, 64, 32, 16, 8, 4, 2, 1):
        if B % cand == 0:
            tb = cand
            break
    out_t = pl.pallas_call(
        _rowwise_body,
        out_shape=jax.ShapeDtypeStruct((1, B), jnp.float32),
        grid=(B // tb,),
        in_specs=[
            pl.BlockSpec((tb, _IN), lambda i: (i, 0)),
            pl.BlockSpec((_H, _PC), lambda i: (0, 0)),
        ],
        out_specs=pl.BlockSpec((1, tb), lambda i: (0, i)),
        compiler_params=pltpu.CompilerParams(
            dimension_semantics=("parallel",),
        ),
    )(x, p)
    return out_t.T
